# baseline (speedup 1.0000x reference)
import functools

import jax
import jax.numpy as jnp
from jax import lax
from jax.experimental import pallas as pl
from jax.experimental.pallas import tpu as pltpu
from jax.experimental.pallas import tpu_sc as plsc

VOCAB = 1000000
EMBED_DIM = 64
RANK = 16


_PCHUNK = 4000
_ASUB = 8


def _half_delta(a_ref, bt):
    a3 = a_ref[...].reshape(RANK, _ASUB, _PCHUNK // _ASUB)
    parts = [
        lax.dot_general(a3[:, u, :], bt, (((0,), (0,)), ((), ())),
                        preferred_element_type=jnp.float32)
        for u in range(_ASUB)
    ]
    return jnp.concatenate(parts, axis=0)


def _fuse_body(a_lo_ref, a_hi_ref, bt_ref, t_ref, o_ref):
    bt = bt_ref[...]
    d_lo = _half_delta(a_lo_ref, bt)
    d_hi = _half_delta(a_hi_ref, bt)
    o_ref[...] = t_ref[...] + jnp.concatenate([d_lo, d_hi], axis=1)


def _fuse_table(lora_A, lora_B, table):
    half = VOCAB // 2
    grid = half // _PCHUNK
    a4 = lora_A.reshape(RANK, 2 * grid, _ASUB, _PCHUNK // _ASUB)
    bt = lora_B.T
    t_lohi = (table.reshape(2, half, EMBED_DIM)
              .transpose(1, 0, 2).reshape(half, 2 * EMBED_DIM))
    return pl.pallas_call(
        _fuse_body,
        grid=(grid,),
        in_specs=[
            pl.BlockSpec((RANK, 1, _ASUB, _PCHUNK // _ASUB),
                         lambda i: (0, i, 0, 0)),
            pl.BlockSpec((RANK, 1, _ASUB, _PCHUNK // _ASUB),
                         lambda i: (0, i + grid, 0, 0)),
            pl.BlockSpec((RANK, EMBED_DIM), lambda i: (0, 0)),
            pl.BlockSpec((_PCHUNK, 2 * EMBED_DIM), lambda i: (i, 0)),
        ],
        out_specs=pl.BlockSpec((_PCHUNK, 2 * EMBED_DIM), lambda i: (i, 0)),
        out_shape=jax.ShapeDtypeStruct((half, 2 * EMBED_DIM), jnp.float32),
    )(a4, a4, bt, t_lohi)


_CHUNK = 128
_K = 5
_NB = 2


def _make_gather(b, s):
    n_tokens = b * s
    info = plsc.get_sparse_core_info()
    nc, ns = info.num_cores, info.num_subcores
    nw = nc * ns
    n_rows = n_tokens // _CHUNK
    rows_per_w = n_rows // nw
    n_batch = rows_per_w // _K
    assert n_tokens % (nw * _CHUNK * _K) == 0
    mesh = plsc.VectorSubcoreMesh(core_axis_name="c", subcore_axis_name="s")

    @functools.partial(
        pl.kernel,
        mesh=mesh,
        compiler_params=pltpu.CompilerParams(use_tc_tiling_on_sc=False),
        out_type=jax.ShapeDtypeStruct((n_rows, _CHUNK, EMBED_DIM),
                                      jnp.float32),
        scratch_types=[
            pltpu.VMEM((_NB, _K, _CHUNK), jnp.int32),
            pltpu.VMEM((_NB, _K, _CHUNK, EMBED_DIM), jnp.float32),
            pltpu.SemaphoreType.DMA((_NB,)),
            pltpu.SemaphoreType.DMA,
            pltpu.SemaphoreType.DMA((_NB,)),
        ],
    )
    def gather(fused_hbm, idx_hbm, out_hbm, idx_v, rows_v, isem, gsem, osem):
        fused2 = fused_hbm
        idx2 = idx_hbm
        out3 = out_hbm
        wid = lax.axis_index("s") * nc + lax.axis_index("c")
        row0 = wid * rows_per_w

        for nb in range(_NB):
            pltpu.async_copy(idx2.at[pl.ds(row0 + nb * _K, _K)],
                             idx_v.at[nb], isem.at[nb])

        def body(g, carry):
            nb = lax.rem(g, _NB)
            r = row0 + g * _K
            pltpu.make_async_copy(idx2.at[pl.ds(r, _K)], idx_v.at[nb],
                                  isem.at[nb]).wait()

            for j in range(_K):
                for i in range(_CHUNK // 16):
                    sl = (nb, j, pl.ds(i * 16, 16))
                    v = idx_v[sl]
                    idx_v[sl] = jnp.where(v < VOCAB // 2, 2 * v,
                                          2 * v - (VOCAB - 1))

            @pl.when(g >= _NB)
            def _():
                pltpu.make_async_copy(rows_v.at[nb],
                                      out3.at[pl.ds(r - _NB * _K, _K)],
                                      osem.at[nb]).wait()

            for j in range(_K):
                pltpu.async_copy(fused2.at[idx_v.at[nb, j]],
                                 rows_v.at[nb, j], gsem)
            for j in range(_K):
                pltpu.make_async_copy(fused2.at[idx_v.at[nb, j]],
                                      rows_v.at[nb, j], gsem).wait()

            @pl.when(g + _NB < n_batch)
            def _():
                pltpu.async_copy(idx2.at[pl.ds(r + _NB * _K, _K)],
                                 idx_v.at[nb], isem.at[nb])

            pltpu.async_copy(rows_v.at[nb], out3.at[pl.ds(r, _K)], osem.at[nb])
            return carry

        lax.fori_loop(0, n_batch, body, 0)

        assert n_batch % _NB == 0
        for nb in range(_NB):
            g = n_batch - _NB + nb
            pltpu.make_async_copy(rows_v.at[nb],
                                  out3.at[pl.ds(row0 + g * _K, _K)],
                                  osem.at[nb]).wait()

    return gather


def _epi_body(i_ref, o_ref):
    v = i_ref[...].reshape(128, 100, 128)
    cols = [v[:, m, :].T for m in range(100)]
    o_ref[...] = jnp.concatenate(cols, axis=0)


def _epilogue(out_lin, b, s):
    n = b * s * EMBED_DIM
    rows = s * EMBED_DIM
    grid = b // 128
    flat = out_lin.reshape(n // 128, 128)
    o = pl.pallas_call(
        _epi_body,
        grid=(grid,),
        in_specs=[pl.BlockSpec((rows, 128), lambda i: (i, 0))],
        out_specs=pl.BlockSpec((rows, 128), lambda i: (0, i)),
        out_shape=jax.ShapeDtypeStruct((rows, b), jnp.float32),
    )(flat)
    return o.reshape(s, EMBED_DIM, b).transpose(2, 0, 1)


def kernel(x, table, lora_A, lora_B):
    fused = _fuse_table(lora_A, lora_B, table)
    fused2 = fused.reshape(VOCAB, EMBED_DIM)
    b, s = x.shape
    x2 = x.astype(jnp.int32).reshape(b * s // _CHUNK, _CHUNK)
    out = _make_gather(b, s)(fused2, x2)
    return _epilogue(out, b, s)


# speedup vs baseline: 1.8746x; 1.8746x over previous
import functools

import jax
import jax.numpy as jnp
from jax import lax
from jax.experimental import pallas as pl
from jax.experimental.pallas import tpu as pltpu
from jax.experimental.pallas import tpu_sc as plsc

VOCAB = 1000000
EMBED_DIM = 64
RANK = 16


_PCHUNK = 4000
_ASUB = 8


def _half_delta(a_ref, bt):
    a3 = a_ref[...].reshape(RANK, _ASUB, _PCHUNK // _ASUB)
    parts = [
        lax.dot_general(a3[:, u, :], bt, (((0,), (0,)), ((), ())),
                        preferred_element_type=jnp.float32)
        for u in range(_ASUB)
    ]
    return jnp.concatenate(parts, axis=0)


def _fuse_body(a_lo_ref, a_hi_ref, bt_ref, t_lo_ref, t_hi_ref, o_ref):
    bt = bt_ref[...]
    d_lo = _half_delta(a_lo_ref, bt)
    d_hi = _half_delta(a_hi_ref, bt)
    o_ref[...] = jnp.concatenate(
        [t_lo_ref[...] + d_lo, t_hi_ref[...] + d_hi], axis=1)


def _fuse_table(lora_A, lora_B, table):
    half = VOCAB // 2
    grid = half // _PCHUNK
    a4 = lora_A.reshape(RANK, 2 * grid, _ASUB, _PCHUNK // _ASUB)
    bt = lora_B.T
    return pl.pallas_call(
        _fuse_body,
        grid=(grid,),
        in_specs=[
            pl.BlockSpec((RANK, 1, _ASUB, _PCHUNK // _ASUB),
                         lambda i: (0, i, 0, 0)),
            pl.BlockSpec((RANK, 1, _ASUB, _PCHUNK // _ASUB),
                         lambda i: (0, i + grid, 0, 0)),
            pl.BlockSpec((RANK, EMBED_DIM), lambda i: (0, 0)),
            pl.BlockSpec((_PCHUNK, EMBED_DIM), lambda i: (i, 0)),
            pl.BlockSpec((_PCHUNK, EMBED_DIM), lambda i: (i + grid, 0)),
        ],
        out_specs=pl.BlockSpec((_PCHUNK, 2 * EMBED_DIM), lambda i: (i, 0)),
        out_shape=jax.ShapeDtypeStruct((half, 2 * EMBED_DIM), jnp.float32),
    )(a4, a4, bt, table, table)


_CHUNK = 128
_K = 5
_NB = 2


def _make_gather(b, s):
    n_tokens = b * s
    info = plsc.get_sparse_core_info()
    nc, ns = info.num_cores, info.num_subcores
    nw = nc * ns
    n_rows = n_tokens // _CHUNK
    rows_per_w = n_rows // nw
    n_batch = rows_per_w // _K
    assert n_tokens % (nw * _CHUNK * _K) == 0
    mesh = plsc.VectorSubcoreMesh(core_axis_name="c", subcore_axis_name="s")

    @functools.partial(
        pl.kernel,
        mesh=mesh,
        compiler_params=pltpu.CompilerParams(use_tc_tiling_on_sc=False),
        out_type=jax.ShapeDtypeStruct((n_rows, _CHUNK, EMBED_DIM),
                                      jnp.float32),
        scratch_types=[
            pltpu.VMEM((_NB, _K, _CHUNK), jnp.int32),
            pltpu.VMEM((_NB, _K, _CHUNK, EMBED_DIM), jnp.float32),
            pltpu.SemaphoreType.DMA((_NB,)),
            pltpu.SemaphoreType.DMA,
            pltpu.SemaphoreType.DMA((_NB,)),
        ],
    )
    def gather(fused_hbm, idx_hbm, out_hbm, idx_v, rows_v, isem, gsem, osem):
        fused2 = fused_hbm
        idx2 = idx_hbm
        out3 = out_hbm
        wid = lax.axis_index("s") * nc + lax.axis_index("c")
        row0 = wid * rows_per_w

        for nb in range(_NB):
            pltpu.async_copy(idx2.at[pl.ds(row0 + nb * _K, _K)],
                             idx_v.at[nb], isem.at[nb])

        def body(g, carry):
            nb = lax.rem(g, _NB)
            r = row0 + g * _K
            pltpu.make_async_copy(idx2.at[pl.ds(r, _K)], idx_v.at[nb],
                                  isem.at[nb]).wait()

            for j in range(_K):
                for i in range(_CHUNK // 16):
                    sl = (nb, j, pl.ds(i * 16, 16))
                    v = idx_v[sl]
                    idx_v[sl] = jnp.where(v < VOCAB // 2, 2 * v,
                                          2 * v - (VOCAB - 1))

            @pl.when(g >= _NB)
            def _():
                pltpu.make_async_copy(rows_v.at[nb],
                                      out3.at[pl.ds(r - _NB * _K, _K)],
                                      osem.at[nb]).wait()

            for j in range(_K):
                pltpu.async_copy(fused2.at[idx_v.at[nb, j]],
                                 rows_v.at[nb, j], gsem)
            for j in range(_K):
                pltpu.make_async_copy(fused2.at[idx_v.at[nb, j]],
                                      rows_v.at[nb, j], gsem).wait()

            @pl.when(g + _NB < n_batch)
            def _():
                pltpu.async_copy(idx2.at[pl.ds(r + _NB * _K, _K)],
                                 idx_v.at[nb], isem.at[nb])

            pltpu.async_copy(rows_v.at[nb], out3.at[pl.ds(r, _K)], osem.at[nb])
            return carry

        lax.fori_loop(0, n_batch, body, 0)

        assert n_batch % _NB == 0
        for nb in range(_NB):
            g = n_batch - _NB + nb
            pltpu.make_async_copy(rows_v.at[nb],
                                  out3.at[pl.ds(row0 + g * _K, _K)],
                                  osem.at[nb]).wait()

    return gather


def _epi_body(i_ref, o_ref):
    v = i_ref[...].reshape(128, 100, 128)
    cols = [v[:, m, :].T for m in range(100)]
    o_ref[...] = jnp.concatenate(cols, axis=0)


def _epilogue(out_lin, b, s):
    n = b * s * EMBED_DIM
    rows = s * EMBED_DIM
    grid = b // 128
    flat = out_lin.reshape(n // 128, 128)
    o = pl.pallas_call(
        _epi_body,
        grid=(grid,),
        in_specs=[pl.BlockSpec((rows, 128), lambda i: (i, 0))],
        out_specs=pl.BlockSpec((rows, 128), lambda i: (0, i)),
        out_shape=jax.ShapeDtypeStruct((rows, b), jnp.float32),
    )(flat)
    return o.reshape(s, EMBED_DIM, b).transpose(2, 0, 1)


def kernel(x, table, lora_A, lora_B):
    fused = _fuse_table(lora_A, lora_B, table)
    fused2 = fused.reshape(VOCAB, EMBED_DIM)
    b, s = x.shape
    x2 = x.astype(jnp.int32).reshape(b * s // _CHUNK, _CHUNK)
    out = _make_gather(b, s)(fused2, x2)
    return _epilogue(out, b, s)


# speedup vs baseline: 3.8288x; 2.0425x over previous
import functools

import jax
import jax.numpy as jnp
from jax import lax
from jax.experimental import pallas as pl
from jax.experimental.pallas import tpu as pltpu
from jax.experimental.pallas import tpu_sc as plsc

VOCAB = 1000000
EMBED_DIM = 64
RANK = 16


_TCH = 8192
_PC = _TCH // 2
_NSTEP = -(-VOCAB // _TCH)
_NPACK = _NSTEP * _PC


def _fuse_body(t_ref, a_ref, bt_ref, o_ref):
    t = t_ref[...]
    a = a_ref[...]
    bt = bt_ref[...]
    dT = lax.dot_general(bt, a, (((0,), (0,)), ((), ())),
                         preferred_element_type=jnp.float32)
    s = t + dT
    st = jnp.concatenate([s[:, :_PC], s[:, _PC:]], axis=0)
    tiles = [st[:, j * 128:(j + 1) * 128].T for j in range(_PC // 128)]
    o_ref[...] = jnp.concatenate(tiles, axis=0)


def _fuse_table(lora_A, lora_B, table):
    return pl.pallas_call(
        _fuse_body,
        grid=(_NSTEP,),
        in_specs=[
            pl.BlockSpec((EMBED_DIM, _TCH), lambda k: (0, k)),
            pl.BlockSpec((RANK, _TCH), lambda k: (0, k)),
            pl.BlockSpec((RANK, EMBED_DIM), lambda k: (0, 0)),
        ],
        out_specs=pl.BlockSpec((_PC, 128), lambda k: (k, 0)),
        out_shape=jax.ShapeDtypeStruct((_NPACK, 128), jnp.float32),
    )(table.T, lora_A, lora_B.T)


_CHUNK = 128
_K = 5
_NB = 2


def _make_gather(b, s):
    n_tokens = b * s
    info = plsc.get_sparse_core_info()
    nc, ns = info.num_cores, info.num_subcores
    nw = nc * ns
    n_rows = n_tokens // _CHUNK
    rows_per_w = n_rows // nw
    n_batch = rows_per_w // _K
    assert n_tokens % (nw * _CHUNK * _K) == 0
    mesh = plsc.VectorSubcoreMesh(core_axis_name="c", subcore_axis_name="s")

    @functools.partial(
        pl.kernel,
        mesh=mesh,
        compiler_params=pltpu.CompilerParams(use_tc_tiling_on_sc=False),
        out_type=jax.ShapeDtypeStruct((n_rows, _CHUNK, EMBED_DIM),
                                      jnp.float32),
        scratch_types=[
            pltpu.VMEM((_NB, _K, _CHUNK), jnp.int32),
            pltpu.VMEM((_NB, _K, _CHUNK, EMBED_DIM), jnp.float32),
            pltpu.SemaphoreType.DMA((_NB,)),
            pltpu.SemaphoreType.DMA,
            pltpu.SemaphoreType.DMA((_NB,)),
        ],
    )
    def gather(fused_hbm, idx_hbm, out_hbm, idx_v, rows_v, isem, gsem, osem):
        fused2 = fused_hbm
        idx2 = idx_hbm
        out3 = out_hbm
        wid = lax.axis_index("s") * nc + lax.axis_index("c")
        row0 = wid * rows_per_w

        for nb in range(_NB):
            pltpu.async_copy(idx2.at[pl.ds(row0 + nb * _K, _K)],
                             idx_v.at[nb], isem.at[nb])

        def body(g, carry):
            nb = lax.rem(g, _NB)
            r = row0 + g * _K
            pltpu.make_async_copy(idx2.at[pl.ds(r, _K)], idx_v.at[nb],
                                  isem.at[nb]).wait()

            for j in range(_K):
                for i in range(_CHUNK // 16):
                    sl = (nb, j, pl.ds(i * 16, 16))
                    v = idx_v[sl]
                    idx_v[sl] = (((v >> 13) << 13)
                                 + ((v & (_PC - 1)) << 1)
                                 + ((v >> 12) & 1))

            @pl.when(g >= _NB)
            def _():
                pltpu.make_async_copy(rows_v.at[nb],
                                      out3.at[pl.ds(r - _NB * _K, _K)],
                                      osem.at[nb]).wait()

            for j in range(_K):
                pltpu.async_copy(fused2.at[idx_v.at[nb, j]],
                                 rows_v.at[nb, j], gsem)
            for j in range(_K):
                pltpu.make_async_copy(fused2.at[idx_v.at[nb, j]],
                                      rows_v.at[nb, j], gsem).wait()

            @pl.when(g + _NB < n_batch)
            def _():
                pltpu.async_copy(idx2.at[pl.ds(r + _NB * _K, _K)],
                                 idx_v.at[nb], isem.at[nb])

            pltpu.async_copy(rows_v.at[nb], out3.at[pl.ds(r, _K)], osem.at[nb])
            return carry

        lax.fori_loop(0, n_batch, body, 0)

        assert n_batch % _NB == 0
        for nb in range(_NB):
            g = n_batch - _NB + nb
            pltpu.make_async_copy(rows_v.at[nb],
                                  out3.at[pl.ds(row0 + g * _K, _K)],
                                  osem.at[nb]).wait()

    return gather


def _epi_body(i_ref, o_ref):
    v = i_ref[...].reshape(128, 100, 128)
    cols = [v[:, m, :].T for m in range(100)]
    o_ref[...] = jnp.concatenate(cols, axis=0)


def _epilogue(out_lin, b, s):
    n = b * s * EMBED_DIM
    rows = s * EMBED_DIM
    grid = b // 128
    flat = out_lin.reshape(n // 128, 128)
    o = pl.pallas_call(
        _epi_body,
        grid=(grid,),
        in_specs=[pl.BlockSpec((rows, 128), lambda i: (i, 0))],
        out_specs=pl.BlockSpec((rows, 128), lambda i: (0, i)),
        out_shape=jax.ShapeDtypeStruct((rows, b), jnp.float32),
    )(flat)
    return o.reshape(s, EMBED_DIM, b).transpose(2, 0, 1)


def kernel(x, table, lora_A, lora_B):
    fused = _fuse_table(lora_A, lora_B, table)
    fused2 = fused.reshape(2 * _NPACK, EMBED_DIM)
    b, s = x.shape
    x2 = x.astype(jnp.int32).reshape(b * s // _CHUNK, _CHUNK)
    out = _make_gather(b, s)(fused2, x2)
    return _epilogue(out, b, s)
